# initial kernel scaffold (unmeasured)
import jax
import jax.numpy as jnp
from jax import lax
from jax.experimental import pallas as pl
from jax.experimental.pallas import tpu as pltpu

N_DEV = 16


def kernel(x, w_mat, scale_x, scale_w):
    m_per, k = x.shape
    n_per = w_mat.shape[1]
    scale = (scale_x * scale_w).astype(jnp.float32).reshape(1, 1)

    def body(x_ref, w_ref, s_ref, out_ref, gather_ref, send_sems, recv_sems):
        my = lax.axis_index("i")
        left = (my - 1) % N_DEV
        right = (my + 1) % N_DEV

        barrier_sem = pltpu.get_barrier_semaphore()
        for nbr in (left, right):
            pl.semaphore_signal(
                barrier_sem, inc=1,
                device_id=(nbr,), device_id_type=pl.DeviceIdType.MESH,
            )
        pl.semaphore_wait(barrier_sem, 2)

        s = s_ref[0, 0]

        def mm_store(chunk, origin):
            acc = lax.dot_general(
                chunk, w_ref[...],
                (((1,), (0,)), ((), ())),
                preferred_element_type=jnp.float32,
            )
            out_ref[pl.ds(origin * m_per, m_per), :] = jnp.maximum(acc * s, 0.0)

        mm_store(x_ref[...], my)

        for h in range(N_DEV - 1):
            src = x_ref if h == 0 else gather_ref.at[h - 1]
            rdma = pltpu.make_async_remote_copy(
                src_ref=src,
                dst_ref=gather_ref.at[h],
                send_sem=send_sems.at[h],
                recv_sem=recv_sems.at[h],
                device_id=(right,),
                device_id_type=pl.DeviceIdType.MESH,
            )
            rdma.start()
            rdma.wait()
            origin = (my - h - 1) % N_DEV
            mm_store(gather_ref[h], origin)

    return pl.pallas_call(
        body,
        out_shape=jax.ShapeDtypeStruct((N_DEV * m_per, n_per), jnp.float32),
        in_specs=[
            pl.BlockSpec(memory_space=pltpu.VMEM),
            pl.BlockSpec(memory_space=pltpu.VMEM),
            pl.BlockSpec(memory_space=pltpu.SMEM),
        ],
        out_specs=pl.BlockSpec(memory_space=pltpu.VMEM),
        scratch_shapes=[
            pltpu.VMEM((N_DEV - 1, m_per, k), x.dtype),
            pltpu.SemaphoreType.DMA((N_DEV - 1,)),
            pltpu.SemaphoreType.DMA((N_DEV - 1,)),
        ],
        compiler_params=pltpu.CompilerParams(collective_id=0),
    )(x, w_mat, scale)


# baseline (device time: 222867 ns/iter reference)
import jax
import jax.numpy as jnp
from jax import lax
from jax.experimental import pallas as pl
from jax.experimental.pallas import tpu as pltpu

N_DEV = 16


def kernel(x, w_mat, scale_x, scale_w):
    m_per, k = x.shape
    n_per = w_mat.shape[1]
    scale = (scale_x * scale_w).astype(jnp.float32).reshape(1, 1)
    x = x.astype(jnp.float8_e4m3fn)
    w_mat = w_mat.astype(jnp.float8_e4m3fn)

    def body(x_ref, w_ref, s_ref, out_ref, gather_ref, send_sems, recv_sems):
        my = lax.axis_index("i")
        left = (my - 1) % N_DEV
        right = (my + 1) % N_DEV

        barrier_sem = pltpu.get_barrier_semaphore()
        for nbr in (left, right):
            pl.semaphore_signal(
                barrier_sem, inc=1,
                device_id=(nbr,), device_id_type=pl.DeviceIdType.MESH,
            )
        pl.semaphore_wait(barrier_sem, 2)

        s = s_ref[0, 0]

        def mm_store(chunk, origin):
            acc = lax.dot_general(
                chunk, w_ref[...],
                (((1,), (0,)), ((), ())),
                preferred_element_type=jnp.float32,
            )
            out_ref[pl.ds(origin * m_per, m_per), :] = jnp.maximum(acc * s, 0.0)

        mm_store(x_ref[...], my)

        for h in range(N_DEV - 1):
            src = x_ref if h == 0 else gather_ref.at[h - 1]
            rdma = pltpu.make_async_remote_copy(
                src_ref=src,
                dst_ref=gather_ref.at[h],
                send_sem=send_sems.at[h],
                recv_sem=recv_sems.at[h],
                device_id=(right,),
                device_id_type=pl.DeviceIdType.MESH,
            )
            rdma.start()
            rdma.wait()
            origin = (my - h - 1) % N_DEV
            mm_store(gather_ref[h], origin)

    return pl.pallas_call(
        body,
        out_shape=jax.ShapeDtypeStruct((N_DEV * m_per, n_per), jnp.float32),
        in_specs=[
            pl.BlockSpec(memory_space=pltpu.VMEM),
            pl.BlockSpec(memory_space=pltpu.VMEM),
            pl.BlockSpec(memory_space=pltpu.SMEM),
        ],
        out_specs=pl.BlockSpec(memory_space=pltpu.VMEM),
        scratch_shapes=[
            pltpu.VMEM((N_DEV - 1, m_per, k), x.dtype),
            pltpu.SemaphoreType.DMA((N_DEV - 1,)),
            pltpu.SemaphoreType.DMA((N_DEV - 1,)),
        ],
        compiler_params=pltpu.CompilerParams(collective_id=0),
    )(x, w_mat, scale)


# device time: 119934 ns/iter; 1.8582x vs baseline; 1.8582x over previous
import jax
import jax.numpy as jnp
from jax import lax
from jax.experimental import pallas as pl
from jax.experimental.pallas import tpu as pltpu

N_DEV = 16
R_HOPS = N_DEV // 2
L_HOPS = N_DEV - 1 - R_HOPS


def kernel(x, w_mat, scale_x, scale_w):
    m_per, k = x.shape
    n_per = w_mat.shape[1]
    scale = (scale_x * scale_w).astype(jnp.float32).reshape(1, 1)
    x = x.astype(jnp.float8_e4m3fn)
    w_mat = w_mat.astype(jnp.float8_e4m3fn)

    def body(x_ref, w_ref, s_ref, out_ref, gr_ref, gl_ref,
             r_send, r_recv, l_send, l_recv):
        my = lax.axis_index("i")
        left = (my - 1) % N_DEV
        right = (my + 1) % N_DEV

        barrier_sem = pltpu.get_barrier_semaphore()
        for nbr in (left, right):
            pl.semaphore_signal(
                barrier_sem, inc=1,
                device_id=(nbr,), device_id_type=pl.DeviceIdType.MESH,
            )
        pl.semaphore_wait(barrier_sem, 2)

        s = s_ref[0, 0]

        def mm_store(chunk, origin):
            acc = lax.dot_general(
                chunk, w_ref[...],
                (((1,), (0,)), ((), ())),
                preferred_element_type=jnp.float32,
            )
            out_ref[pl.ds(origin * m_per, m_per), :] = jnp.maximum(acc * s, 0.0)

        def mk(h, slots, send_sems, recv_sems, tgt):
            return pltpu.make_async_remote_copy(
                src_ref=x_ref if h == 0 else slots.at[h - 1],
                dst_ref=slots.at[h],
                send_sem=send_sems.at[h],
                recv_sem=recv_sems.at[h],
                device_id=(tgt,),
                device_id_type=pl.DeviceIdType.MESH,
            )

        rs = [mk(h, gr_ref, r_send, r_recv, right) for h in range(R_HOPS)]
        ls = [mk(h, gl_ref, l_send, l_recv, left) for h in range(L_HOPS)]

        rs[0].start()
        ls[0].start()
        mm_store(x_ref[...], my)

        for h in range(R_HOPS):
            rs[h].wait_recv()
            if h + 1 < R_HOPS:
                rs[h + 1].start()
            if h < L_HOPS:
                ls[h].wait_recv()
                if h + 1 < L_HOPS:
                    ls[h + 1].start()
            mm_store(gr_ref[h], (my - h - 1) % N_DEV)
            if h < L_HOPS:
                mm_store(gl_ref[h], (my + h + 1) % N_DEV)

        for r in rs + ls:
            r.wait_send()

    return pl.pallas_call(
        body,
        out_shape=jax.ShapeDtypeStruct((N_DEV * m_per, n_per), jnp.float32),
        in_specs=[
            pl.BlockSpec(memory_space=pltpu.VMEM),
            pl.BlockSpec(memory_space=pltpu.VMEM),
            pl.BlockSpec(memory_space=pltpu.SMEM),
        ],
        out_specs=pl.BlockSpec(memory_space=pltpu.VMEM),
        scratch_shapes=[
            pltpu.VMEM((R_HOPS, m_per, k), x.dtype),
            pltpu.VMEM((L_HOPS, m_per, k), x.dtype),
            pltpu.SemaphoreType.DMA((R_HOPS,)),
            pltpu.SemaphoreType.DMA((R_HOPS,)),
            pltpu.SemaphoreType.DMA((L_HOPS,)),
            pltpu.SemaphoreType.DMA((L_HOPS,)),
        ],
        compiler_params=pltpu.CompilerParams(collective_id=0),
    )(x, w_mat, scale)


# device time: 104778 ns/iter; 2.1270x vs baseline; 1.1446x over previous
import jax
import jax.numpy as jnp
from jax import lax
from jax.experimental import pallas as pl
from jax.experimental.pallas import tpu as pltpu

N_DEV = 16
HOPS = N_DEV // 2


def kernel(x, w_mat, scale_x, scale_w):
    m_per, k = x.shape
    n_per = w_mat.shape[1]
    half = m_per // 2
    scale = (scale_x * scale_w).astype(jnp.float32).reshape(1, 1)
    x = x.astype(jnp.float8_e4m3fn)
    w_mat = w_mat.astype(jnp.float8_e4m3fn)

    def body(x_ref, w_ref, s_ref, out_ref, gr_ref, gl_ref,
             r_send, r_recv, l_send, l_recv):
        my = lax.axis_index("i")
        left = (my - 1) % N_DEV
        right = (my + 1) % N_DEV

        barrier_sem = pltpu.get_barrier_semaphore()
        for nbr in (left, right):
            pl.semaphore_signal(
                barrier_sem, inc=1,
                device_id=(nbr,), device_id_type=pl.DeviceIdType.MESH,
            )
        pl.semaphore_wait(barrier_sem, 2)

        s = s_ref[0, 0]

        def mm_store(chunk, row0, rows):
            acc = lax.dot_general(
                chunk, w_ref[...],
                (((1,), (0,)), ((), ())),
                preferred_element_type=jnp.float32,
            )
            out_ref[pl.ds(row0, rows), :] = jnp.maximum(acc * s, 0.0)

        def mk(h, j, slots, send_sems, recv_sems, tgt):
            rows = pl.ds(j * half, half)
            src = x_ref if h == 0 else slots.at[h - 1]
            return pltpu.make_async_remote_copy(
                src_ref=src.at[rows, :],
                dst_ref=slots.at[h, rows, :],
                send_sem=send_sems.at[h, j],
                recv_sem=recv_sems.at[h, j],
                device_id=(tgt,),
                device_id_type=pl.DeviceIdType.MESH,
            )

        def halves(h):
            if h < HOPS - 1:
                return (0, 1), (0, 1)
            return (0,), (1,)

        rs = {(h, j): mk(h, j, gr_ref, r_send, r_recv, right)
              for h in range(HOPS) for j in halves(h)[0]}
        ls = {(h, j): mk(h, j, gl_ref, l_send, l_recv, left)
              for h in range(HOPS) for j in halves(h)[1]}

        for j in (0, 1):
            rs[(0, j)].start()
            ls[(0, j)].start()
        mm_store(x_ref[...], my * m_per, m_per)

        for h in range(HOPS):
            rj, lj = halves(h)
            for j in rj:
                rs[(h, j)].wait_recv()
                if (h + 1, j) in rs:
                    rs[(h + 1, j)].start()
            for j in lj:
                ls[(h, j)].wait_recv()
                if (h + 1, j) in ls:
                    ls[(h + 1, j)].start()
            if h < HOPS - 1:
                mm_store(gr_ref[h], ((my - h - 1) % N_DEV) * m_per, m_per)
                mm_store(gl_ref[h], ((my + h + 1) % N_DEV) * m_per, m_per)
            else:
                anti = (my + HOPS) % N_DEV
                mm_store(gr_ref[h, :half, :], anti * m_per, half)
                mm_store(gl_ref[h, half:, :], anti * m_per + half, half)

        for r in list(rs.values()) + list(ls.values()):
            r.wait_send()

    return pl.pallas_call(
        body,
        out_shape=jax.ShapeDtypeStruct((N_DEV * m_per, n_per), jnp.float32),
        in_specs=[
            pl.BlockSpec(memory_space=pltpu.VMEM),
            pl.BlockSpec(memory_space=pltpu.VMEM),
            pl.BlockSpec(memory_space=pltpu.SMEM),
        ],
        out_specs=pl.BlockSpec(memory_space=pltpu.VMEM),
        scratch_shapes=[
            pltpu.VMEM((HOPS, m_per, k), x.dtype),
            pltpu.VMEM((HOPS, m_per, k), x.dtype),
            pltpu.SemaphoreType.DMA((HOPS, 2)),
            pltpu.SemaphoreType.DMA((HOPS, 2)),
            pltpu.SemaphoreType.DMA((HOPS, 2)),
            pltpu.SemaphoreType.DMA((HOPS, 2)),
        ],
        compiler_params=pltpu.CompilerParams(collective_id=0),
    )(x, w_mat, scale)


# device time: 103708 ns/iter; 2.1490x vs baseline; 1.0103x over previous
import jax
import jax.numpy as jnp
from jax import lax
from jax.experimental import pallas as pl
from jax.experimental.pallas import tpu as pltpu

N_DEV = 16
HOPS = N_DEV // 2
XP = 4
WP = 8


def kernel(x, w_mat, scale_x, scale_w):
    m_per, k = x.shape
    n_per = w_mat.shape[1]
    half = m_per // 2
    xrows = m_per // XP
    wrows = w_mat.shape[0] // WP
    scale = (scale_x * scale_w).astype(jnp.float32).reshape(1, 1)

    def body(x_hbm, w_hbm, s_ref, out_ref, x8, w8, gr_ref, gl_ref,
             sx, sw, sx_sems, sw_sems, r_send, r_recv, l_send, l_recv):
        my = lax.axis_index("i")
        left = (my - 1) % N_DEV
        right = (my + 1) % N_DEV

        def stage(src_hbm, p, rows, buf, sems):
            return pltpu.make_async_copy(
                src_hbm.at[pl.ds(p * rows, rows), :], buf.at[p % 2], sems.at[p]
            )

        for p in range(min(2, XP)):
            stage(x_hbm, p, xrows, sx, sx_sems).start()

        barrier_sem = pltpu.get_barrier_semaphore()
        for nbr in (left, right):
            pl.semaphore_signal(
                barrier_sem, inc=1,
                device_id=(nbr,), device_id_type=pl.DeviceIdType.MESH,
            )
        pl.semaphore_wait(barrier_sem, 2)

        s = s_ref[0, 0]

        def mm_store(chunk, row0, rows):
            acc = lax.dot_general(
                chunk, w8[...],
                (((1,), (0,)), ((), ())),
                preferred_element_type=jnp.float32,
            )
            out_ref[pl.ds(row0, rows), :] = jnp.maximum(acc * s, 0.0)

        def mk(h, j, slots, send_sems, recv_sems, tgt):
            rows = pl.ds(j * half, half)
            src = x8 if h == 0 else slots.at[h - 1]
            return pltpu.make_async_remote_copy(
                src_ref=src.at[rows, :],
                dst_ref=slots.at[h, rows, :],
                send_sem=send_sems.at[h, j],
                recv_sem=recv_sems.at[h, j],
                device_id=(tgt,),
                device_id_type=pl.DeviceIdType.MESH,
            )

        def halves(h):
            if h < HOPS - 1:
                return (0, 1), (0, 1)
            return (0,), (1,)

        rs = {(h, j): mk(h, j, gr_ref, r_send, r_recv, right)
              for h in range(HOPS) for j in halves(h)[0]}
        ls = {(h, j): mk(h, j, gl_ref, l_send, l_recv, left)
              for h in range(HOPS) for j in halves(h)[1]}

        for p in range(XP):
            stage(x_hbm, p, xrows, sx, sx_sems).wait()
            x8[pl.ds(p * xrows, xrows), :] = sx[p % 2].astype(jnp.float8_e4m3fn)
            if p + 2 < XP:
                stage(x_hbm, p + 2, xrows, sx, sx_sems).start()
            if (p + 1) * xrows == half:
                rs[(0, 0)].start()
                ls[(0, 0)].start()
            elif (p + 1) * xrows == m_per:
                rs[(0, 1)].start()
                ls[(0, 1)].start()

        for p in range(min(2, WP)):
            stage(w_hbm, p, wrows, sw, sw_sems).start()
        for p in range(WP):
            stage(w_hbm, p, wrows, sw, sw_sems).wait()
            w8[pl.ds(p * wrows, wrows), :] = sw[p % 2].astype(jnp.float8_e4m3fn)
            if p + 2 < WP:
                stage(w_hbm, p + 2, wrows, sw, sw_sems).start()

        mm_store(x8[...], my * m_per, m_per)

        for h in range(HOPS):
            rj, lj = halves(h)
            for j in rj:
                rs[(h, j)].wait_recv()
                if (h + 1, j) in rs:
                    rs[(h + 1, j)].start()
            for j in lj:
                ls[(h, j)].wait_recv()
                if (h + 1, j) in ls:
                    ls[(h + 1, j)].start()
            if h < HOPS - 1:
                mm_store(gr_ref[h], ((my - h - 1) % N_DEV) * m_per, m_per)
                mm_store(gl_ref[h], ((my + h + 1) % N_DEV) * m_per, m_per)
            else:
                anti = (my + HOPS) % N_DEV
                mm_store(gr_ref[h, :half, :], anti * m_per, half)
                mm_store(gl_ref[h, half:, :], anti * m_per + half, half)

        for r in list(rs.values()) + list(ls.values()):
            r.wait_send()

    fp8 = jnp.float8_e4m3fn
    return pl.pallas_call(
        body,
        out_shape=jax.ShapeDtypeStruct((N_DEV * m_per, n_per), jnp.float32),
        in_specs=[
            pl.BlockSpec(memory_space=pltpu.MemorySpace.HBM),
            pl.BlockSpec(memory_space=pltpu.MemorySpace.HBM),
            pl.BlockSpec(memory_space=pltpu.SMEM),
        ],
        out_specs=pl.BlockSpec(memory_space=pltpu.VMEM),
        scratch_shapes=[
            pltpu.VMEM((m_per, k), fp8),
            pltpu.VMEM((w_mat.shape[0], n_per), fp8),
            pltpu.VMEM((HOPS, m_per, k), fp8),
            pltpu.VMEM((HOPS, m_per, k), fp8),
            pltpu.VMEM((2, xrows, k), jnp.float32),
            pltpu.VMEM((2, wrows, n_per), jnp.float32),
            pltpu.SemaphoreType.DMA((XP,)),
            pltpu.SemaphoreType.DMA((WP,)),
            pltpu.SemaphoreType.DMA((HOPS, 2)),
            pltpu.SemaphoreType.DMA((HOPS, 2)),
            pltpu.SemaphoreType.DMA((HOPS, 2)),
            pltpu.SemaphoreType.DMA((HOPS, 2)),
        ],
        compiler_params=pltpu.CompilerParams(collective_id=0),
    )(x, w_mat, scale)
